# baseline (speedup 1.0000x reference)
import functools

import jax
import jax.numpy as jnp
from jax import lax
from jax.experimental import pallas as pl
from jax.experimental.pallas import tpu as pltpu
from jax.experimental.pallas import tpu_sc as plsc

_NUM_CORES = 2
_NCHUNK = 4


@functools.partial(jax.jit, static_argnames=("n",))
def _sc_copy(x_flat, n):
    per_w = n // _NUM_CORES
    chunk = per_w // _NCHUNK

    def body(x_hbm, out_hbm, *scratch):
        bufs = scratch[:_NCHUNK]
        in_sems = scratch[_NCHUNK:2 * _NCHUNK]
        out_sems = scratch[2 * _NCHUNK:]
        base = lax.axis_index("c") * per_w
        loads = []
        for i in range(_NCHUNK):
            loads.append(pltpu.async_copy(
                x_hbm.at[pl.ds(base + i * chunk, chunk)], bufs[i],
                in_sems[i]))
        stores = []
        for i in range(_NCHUNK):
            loads[i].wait()
            stores.append(pltpu.async_copy(
                bufs[i], out_hbm.at[pl.ds(base + i * chunk, chunk)],
                out_sems[i]))
        for s in stores:
            s.wait()

    mesh = plsc.ScalarSubcoreMesh(axis_name="c", num_cores=_NUM_CORES)
    return pl.kernel(
        body,
        out_type=jax.ShapeDtypeStruct((n,), jnp.float32),
        mesh=mesh,
        scratch_types=(
            [pltpu.VMEM_SHARED((chunk,), jnp.float32)] * _NCHUNK
            + [pltpu.SemaphoreType.DMA] * (2 * _NCHUNK)
        ),
    )(x_flat)


def kernel(x, buffer):
    n = x.size
    out = _sc_copy(x.reshape(n), n)
    return out.reshape(x.shape)


# speedup vs baseline: 4.3386x; 4.3386x over previous
import functools

import jax
import jax.numpy as jnp
from jax.experimental import pallas as pl
from jax.experimental.pallas import tpu as pltpu


@functools.partial(jax.jit, static_argnames=("n",))
def _tc_copy(x_flat, n):
    rows = n // 1280
    grid = 10

    def body(x_ref, o_ref):
        o_ref[...] = x_ref[...]

    return pl.pallas_call(
        body,
        grid=(grid,),
        in_specs=[pl.BlockSpec((rows // grid, 1280), lambda i: (i, 0))],
        out_specs=pl.BlockSpec((rows // grid, 1280), lambda i: (i, 0)),
        out_shape=jax.ShapeDtypeStruct((rows, 1280), jnp.float32),
    )(x_flat.reshape(rows, 1280))


def kernel(x, buffer):
    n = x.size
    out = _tc_copy(x.reshape(n), n)
    return out.reshape(x.shape)
